# baseline (speedup 1.0000x reference)
import jax
import jax.numpy as jnp
from jax.experimental import pallas as pl
from jax.experimental.pallas import tpu as pltpu

_B, _C, _N, _T = 8, 32, 1024, 12
_CT = _C * _T
_BLK = 256


def _dot(a, b, dims):
    return jax.lax.dot_general(a, b, (dims, ((), ())),
                               preferred_element_type=jnp.float32)


def _stage1(xr_ref, adj_ref, wrx_ref, wrx1_ref, w0_ref, b0_ref, bc_ref,
            z1_ref, s_ref):
    b = pl.program_id(0)

    @pl.when(b == 0)
    def _():
        s_ref[...] = jnp.zeros_like(s_ref)

    xb = xr_ref[0]
    x1b = _dot(adj_ref[...], xb, ((0,), (0,)))
    sx = _dot(xb, wrx_ref[...], ((1,), (0,)))
    sx1 = _dot(x1b.astype(jnp.bfloat16), wrx1_ref[...], ((1,), (0,)))
    s_ref[...] += sx + sx1

    @pl.when(b == _B - 1)
    def _():
        s = s_ref[...] + (_B * _T) * bc_ref[...][None, :]
        z1 = _dot(s.astype(jnp.bfloat16), w0_ref[...], ((1,), (1,)))
        z1_ref[...] = jnp.tanh(z1 + b0_ref[...][None, :]).astype(jnp.bfloat16)


def _stage2(z1_ref, w1_ref, b1_ref, w2_ref, b2_ref, g_ref, y_ref):
    i = pl.program_id(0)
    z2 = _dot(z1_ref[...], w1_ref[...], ((1,), (1,)))
    z2 = jnp.tanh(z2 + b1_ref[...][None, :]).astype(jnp.bfloat16)
    z3 = _dot(z2, w2_ref[...], ((1,), (1,)))
    a = jnp.tanh(z3 + b2_ref[...][None, :]) + g_ref[...]
    m = jnp.max(a, axis=1, keepdims=True)
    cols = jax.lax.broadcasted_iota(jnp.int32, a.shape, 1)
    k = jnp.min(jnp.where(a == m, cols, _N), axis=1, keepdims=True)
    rows = i * _BLK + jax.lax.broadcasted_iota(jnp.int32, a.shape, 0)
    y_ref[...] = jnp.where((cols == k) & (cols != rows),
                           jnp.float32(1.0), jnp.float32(0.0))


def kernel(x, adj, W_conv, b_conv, W0, b0, W1, b1, W2, b2):
    bf = lambda a: a.astype(jnp.bfloat16)
    xr = bf(jnp.transpose(x, (0, 2, 1, 3)).reshape(_B, _N, _CT))
    wrx = bf(jnp.repeat(W_conv[:, :_C].T, _T, axis=0))
    wrx1 = bf(jnp.repeat(W_conv[:, _C:].T, _T, axis=0))
    U = jax.random.uniform(jax.random.key(42), (_N, _N), dtype=jnp.float32)
    g = -jnp.log(-jnp.log(U + 1e-10) + 1e-10)

    z1 = pl.pallas_call(
        _stage1,
        grid=(_B,),
        in_specs=[
            pl.BlockSpec((1, _N, _CT), lambda b: (b, 0, 0)),
            pl.BlockSpec((_N, _N), lambda b: (0, 0)),
            pl.BlockSpec((_CT, _C), lambda b: (0, 0)),
            pl.BlockSpec((_CT, _C), lambda b: (0, 0)),
            pl.BlockSpec((_N, _C), lambda b: (0, 0)),
            pl.BlockSpec((_N,), lambda b: (0,)),
            pl.BlockSpec((_C,), lambda b: (0,)),
        ],
        out_specs=pl.BlockSpec((_N, _N), lambda b: (0, 0)),
        out_shape=jax.ShapeDtypeStruct((_N, _N), jnp.bfloat16),
        scratch_shapes=[pltpu.VMEM((_N, _C), jnp.float32)],
    )(xr, bf(adj), wrx, wrx1, bf(W0), b0, b_conv)

    y = pl.pallas_call(
        _stage2,
        grid=(_N // _BLK,),
        in_specs=[
            pl.BlockSpec((_BLK, _N), lambda i: (i, 0)),
            pl.BlockSpec((2 * _N, _N), lambda i: (0, 0)),
            pl.BlockSpec((2 * _N,), lambda i: (0,)),
            pl.BlockSpec((_N, 2 * _N), lambda i: (0, 0)),
            pl.BlockSpec((_N,), lambda i: (0,)),
            pl.BlockSpec((_BLK, _N), lambda i: (i, 0)),
        ],
        out_specs=pl.BlockSpec((_BLK, _N), lambda i: (i, 0)),
        out_shape=jax.ShapeDtypeStruct((_N, _N), jnp.float32),
    )(z1, bf(W1), b1, bf(W2), b2, g)
    return y


# speedup vs baseline: 1.4154x; 1.4154x over previous
import jax
import jax.numpy as jnp
import numpy as np
from jax.experimental import pallas as pl
from jax.experimental.pallas import tpu as pltpu

_B, _C, _N, _T = 8, 32, 1024, 12
_CT = _C * _T
_BLK = 256

_G_CONST = np.asarray(
    -jnp.log(-jnp.log(jax.random.uniform(jax.random.key(42), (_N, _N),
                                         dtype=jnp.float32) + 1e-10) + 1e-10))


def _dot(a, b, dims):
    return jax.lax.dot_general(a, b, (dims, ((), ())),
                               preferred_element_type=jnp.float32)


def _stage1(xr_ref, adj_ref, wrx_ref, wrx1_ref, w0_ref, b0_ref, bc_ref,
            z1_ref, s_ref, adjbf_ref):
    b = pl.program_id(0)

    @pl.when(b == 0)
    def _():
        s_ref[...] = jnp.zeros_like(s_ref)
        adjbf_ref[...] = adj_ref[...].astype(jnp.bfloat16)

    xb = xr_ref[0]
    x1b = _dot(adjbf_ref[...], xb, ((0,), (0,)))
    sx = _dot(xb, wrx_ref[...], ((1,), (0,)))
    sx1 = _dot(x1b.astype(jnp.bfloat16), wrx1_ref[...], ((1,), (0,)))
    s_ref[...] += sx + sx1

    @pl.when(b == _B - 1)
    def _():
        s = s_ref[...] + (_B * _T) * bc_ref[...][None, :]
        z1 = _dot(s.astype(jnp.bfloat16), w0_ref[...].astype(jnp.bfloat16),
                  ((1,), (1,)))
        z1_ref[...] = jnp.tanh(z1 + b0_ref[...][None, :]).astype(jnp.bfloat16)


def _stage2(z1_ref, w1_ref, b1_ref, w2_ref, b2_ref, g_ref, y_ref,
            w1bf_ref, w2bf_ref):
    i = pl.program_id(0)

    @pl.when(i == 0)
    def _():
        w1bf_ref[...] = w1_ref[...].astype(jnp.bfloat16)
        w2bf_ref[...] = w2_ref[...].astype(jnp.bfloat16)

    z2 = _dot(z1_ref[...], w1bf_ref[...], ((1,), (1,)))
    z2 = jnp.tanh(z2 + b1_ref[...][None, :]).astype(jnp.bfloat16)
    z3 = _dot(z2, w2bf_ref[...], ((1,), (1,)))
    a = jnp.tanh(z3 + b2_ref[...][None, :]) + g_ref[...]
    m = jnp.max(a, axis=1, keepdims=True)
    cols = jax.lax.broadcasted_iota(jnp.int32, a.shape, 1)
    k = jnp.min(jnp.where(a == m, cols, _N), axis=1, keepdims=True)
    rows = i * _BLK + jax.lax.broadcasted_iota(jnp.int32, a.shape, 0)
    y_ref[...] = jnp.where((cols == k) & (cols != rows),
                           jnp.float32(1.0), jnp.float32(0.0))


def kernel(x, adj, W_conv, b_conv, W0, b0, W1, b1, W2, b2):
    bf = lambda a: a.astype(jnp.bfloat16)
    xr = bf(jnp.transpose(x, (0, 2, 1, 3)).reshape(_B, _N, _CT))
    wrx = bf(jnp.repeat(W_conv[:, :_C].T, _T, axis=0))
    wrx1 = bf(jnp.repeat(W_conv[:, _C:].T, _T, axis=0))
    g = jnp.asarray(_G_CONST)

    z1 = pl.pallas_call(
        _stage1,
        grid=(_B,),
        in_specs=[
            pl.BlockSpec((1, _N, _CT), lambda b: (b, 0, 0)),
            pl.BlockSpec((_N, _N), lambda b: (0, 0)),
            pl.BlockSpec((_CT, _C), lambda b: (0, 0)),
            pl.BlockSpec((_CT, _C), lambda b: (0, 0)),
            pl.BlockSpec((_N, _C), lambda b: (0, 0)),
            pl.BlockSpec((_N,), lambda b: (0,)),
            pl.BlockSpec((_C,), lambda b: (0,)),
        ],
        out_specs=pl.BlockSpec((_N, _N), lambda b: (0, 0)),
        out_shape=jax.ShapeDtypeStruct((_N, _N), jnp.bfloat16),
        scratch_shapes=[pltpu.VMEM((_N, _C), jnp.float32),
                        pltpu.VMEM((_N, _N), jnp.bfloat16)],
    )(xr, adj, wrx, wrx1, W0, b0, b_conv)

    y = pl.pallas_call(
        _stage2,
        grid=(_N // _BLK,),
        in_specs=[
            pl.BlockSpec((_BLK, _N), lambda i: (i, 0)),
            pl.BlockSpec((2 * _N, _N), lambda i: (0, 0)),
            pl.BlockSpec((2 * _N,), lambda i: (0,)),
            pl.BlockSpec((_N, 2 * _N), lambda i: (0, 0)),
            pl.BlockSpec((_N,), lambda i: (0,)),
            pl.BlockSpec((_BLK, _N), lambda i: (i, 0)),
        ],
        out_specs=pl.BlockSpec((_BLK, _N), lambda i: (i, 0)),
        out_shape=jax.ShapeDtypeStruct((_N, _N), jnp.float32),
        scratch_shapes=[pltpu.VMEM((2 * _N, _N), jnp.bfloat16),
                        pltpu.VMEM((_N, 2 * _N), jnp.bfloat16)],
    )(z1, W1, b1, W2, b2, g)
    return y


# speedup vs baseline: 1.4525x; 1.0262x over previous
import jax
import jax.numpy as jnp
import numpy as np
from jax.experimental import pallas as pl
from jax.experimental.pallas import tpu as pltpu

_B, _C, _N, _T = 8, 32, 1024, 12
_CT = _C * _T
_BLK = 256
_NBLK = _N // _BLK

_G_CONST = np.asarray(
    -jnp.log(-jnp.log(jax.random.uniform(jax.random.key(42), (_N, _N),
                                         dtype=jnp.float32) + 1e-10) + 1e-10))


def _dot(a, b, dims):
    return jax.lax.dot_general(a, b, (dims, ((), ())),
                               preferred_element_type=jnp.float32)


def _fused(xr_ref, adj_ref, wrx_ref, wrx1_ref, w0_ref, b0_ref, bc_ref,
           w1_ref, b1_ref, w2_ref, b2_ref, g_ref, y_ref,
           s_ref, adjbf_ref, z1_ref, w1bf_ref, w2bf_ref):
    step = pl.program_id(0)

    @pl.when(step == 0)
    def _():
        s_ref[...] = jnp.zeros_like(s_ref)
        adjbf_ref[...] = adj_ref[...].astype(jnp.bfloat16)
        w1bf_ref[...] = w1_ref[...].astype(jnp.bfloat16)
        w2bf_ref[...] = w2_ref[...].astype(jnp.bfloat16)

    @pl.when(step < _B)
    def _():
        xb = xr_ref[0]
        x1b = _dot(adjbf_ref[...], xb, ((0,), (0,)))
        sx = _dot(xb, wrx_ref[...], ((1,), (0,)))
        sx1 = _dot(x1b.astype(jnp.bfloat16), wrx1_ref[...], ((1,), (0,)))
        s_ref[...] += sx + sx1

    @pl.when(step == _B - 1)
    def _():
        s = s_ref[...] + (_B * _T) * bc_ref[...][None, :]
        z1 = _dot(s.astype(jnp.bfloat16), w0_ref[...].astype(jnp.bfloat16),
                  ((1,), (1,)))
        z1_ref[...] = jnp.tanh(z1 + b0_ref[...][None, :]).astype(jnp.bfloat16)

    @pl.when(step >= _B)
    def _():
        i = step - _B
        z1 = z1_ref[pl.ds(i * _BLK, _BLK), :]
        z2 = _dot(z1, w1bf_ref[...], ((1,), (1,)))
        z2 = jnp.tanh(z2 + b1_ref[...][None, :]).astype(jnp.bfloat16)
        z3 = _dot(z2, w2bf_ref[...], ((1,), (1,)))
        a = jnp.tanh(z3 + b2_ref[...][None, :]) + g_ref[...]
        m = jnp.max(a, axis=1, keepdims=True)
        cols = jax.lax.broadcasted_iota(jnp.int32, a.shape, 1)
        k = jnp.min(jnp.where(a == m, cols, _N), axis=1, keepdims=True)
        rows = i * _BLK + jax.lax.broadcasted_iota(jnp.int32, a.shape, 0)
        y_ref[...] = jnp.where((cols == k) & (cols != rows),
                               jnp.float32(1.0), jnp.float32(0.0))


def kernel(x, adj, W_conv, b_conv, W0, b0, W1, b1, W2, b2):
    bf = lambda a: a.astype(jnp.bfloat16)
    xr = bf(jnp.transpose(x, (0, 2, 1, 3)).reshape(_B, _N, _CT))
    wrx = bf(jnp.repeat(W_conv[:, :_C].T, _T, axis=0))
    wrx1 = bf(jnp.repeat(W_conv[:, _C:].T, _T, axis=0))
    g = jnp.asarray(_G_CONST)

    y = pl.pallas_call(
        _fused,
        grid=(_B + _NBLK,),
        in_specs=[
            pl.BlockSpec((1, _N, _CT), lambda s: (min_idx(s, _B - 1), 0, 0)),
            pl.BlockSpec((_N, _N), lambda s: (0, 0)),
            pl.BlockSpec((_CT, _C), lambda s: (0, 0)),
            pl.BlockSpec((_CT, _C), lambda s: (0, 0)),
            pl.BlockSpec((_N, _C), lambda s: (0, 0)),
            pl.BlockSpec((_N,), lambda s: (0,)),
            pl.BlockSpec((_C,), lambda s: (0,)),
            pl.BlockSpec((2 * _N, _N), lambda s: (0, 0)),
            pl.BlockSpec((2 * _N,), lambda s: (0,)),
            pl.BlockSpec((_N, 2 * _N), lambda s: (0, 0)),
            pl.BlockSpec((_N,), lambda s: (0,)),
            pl.BlockSpec((_BLK, _N), lambda s: (max_idx(s - _B, 0), 0)),
        ],
        out_specs=pl.BlockSpec((_BLK, _N), lambda s: (max_idx(s - _B, 0), 0)),
        out_shape=jax.ShapeDtypeStruct((_N, _N), jnp.float32),
        scratch_shapes=[
            pltpu.VMEM((_N, _C), jnp.float32),
            pltpu.VMEM((_N, _N), jnp.bfloat16),
            pltpu.VMEM((_N, _N), jnp.bfloat16),
            pltpu.VMEM((2 * _N, _N), jnp.bfloat16),
            pltpu.VMEM((_N, 2 * _N), jnp.bfloat16),
        ],
    )(xr, adj, wrx, wrx1, W0, b0, b_conv, W1, b1, W2, b2, g)
    return y


def min_idx(a, b):
    return jnp.minimum(a, b)


def max_idx(a, b):
    return jnp.maximum(a, b)
